# Initial kernel scaffold; baseline (speedup 1.0000x reference)
import jax
import jax.numpy as jnp
from jax.experimental import pallas as pl


def kernel(x, params):
    raise NotImplementedError("write your pallas kernel here")


# speedup vs baseline: 320.3005x; 320.3005x over previous
import functools

import jax
import jax.numpy as jnp
from jax import lax
from jax.experimental import pallas as pl
from jax.experimental.pallas import tpu as pltpu
from jax.experimental.pallas import tpu_sc as plsc

D_MODEL = 768
N_HEADS = 12
DH = 64
DP = 2 * DH
BUCKET = 64
N_HASHES = 2
D_FFN = 3072

NC = 2
NS = 16
NW = NC * NS

T_BLK = 512
TILE_CH = 4


def _qkv_body(h_ref, wpack_ref, qkv_ref):
    qkv_ref[...] = jnp.dot(h_ref[...].astype(jnp.bfloat16), wpack_ref[...],
                           preferred_element_type=jnp.float32)


def _qkv_call(h2, wpack):
    bt = h2.shape[0]
    grid = (bt // T_BLK,)
    return pl.pallas_call(
        _qkv_body,
        grid=grid,
        in_specs=[
            pl.BlockSpec((T_BLK, D_MODEL), lambda i: (i, 0)),
            pl.BlockSpec((D_MODEL, N_HEADS * DP), lambda i: (0, 0)),
        ],
        out_specs=pl.BlockSpec((T_BLK, N_HEADS * DP), lambda i: (i, 0)),
        out_shape=jax.ShapeDtypeStruct((bt, N_HEADS * DP), jnp.float32),
    )(h2, wpack)


def _sc_mesh():
    return plsc.VectorSubcoreMesh(core_axis_name="c", subcore_axis_name="s",
                                  num_cores=NC, num_subcores=NS)

_SC_CH = 1024
_SC_SUB = _SC_CH // 128
_SC_HALF = _SC_CH // 2


def _sc_gather(tbl2, idx2):
    nrows = idx2.shape[0] * idx2.shape[1]
    rows_w = nrows // NW
    nch = rows_w // _SC_CH

    @functools.partial(
        pl.kernel,
        out_type=jax.ShapeDtypeStruct((nrows, DP), jnp.float32),
        mesh=_sc_mesh(),
        scratch_types=[
            pltpu.VMEM((_SC_SUB, 128), jnp.int32),
            pltpu.VMEM((_SC_HALF, DP), jnp.float32),
            pltpu.SemaphoreType.DMA,
        ],
    )
    def gk(tbl_hbm, idx_hbm, out_hbm, idx_v, buf_v, s1):
        wid = lax.axis_index("s") * NC + lax.axis_index("c")
        base = wid * rows_w

        def body(i, carry):
            off = pl.multiple_of(base + i * _SC_CH, _SC_CH)
            pltpu.sync_copy(
                idx_hbm.at[pl.ds(pl.multiple_of(off // 128, _SC_SUB),
                                 _SC_SUB)], idx_v)
            for half in range(2):
                cps = []
                for j in range(_SC_SUB // 2):
                    cps.append(pltpu.async_copy(
                        tbl_hbm.at[idx_v.at[half * (_SC_SUB // 2) + j]],
                        buf_v.at[pl.ds(j * 128, 128)], s1))
                for cp in cps:
                    cp.wait()
                pltpu.sync_copy(
                    buf_v, out_hbm.at[pl.ds(off + half * _SC_HALF,
                                            _SC_HALF)])
            return carry

        lax.fori_loop(0, nch, body, 0)

    return gk(tbl2, idx2)


def _sc_scatter(tbl2, dst2):
    nrows = dst2.shape[0] * dst2.shape[1]
    rows_w = nrows // NW
    nch = rows_w // _SC_CH

    @functools.partial(
        pl.kernel,
        out_type=jax.ShapeDtypeStruct((nrows, DP), jnp.float32),
        mesh=_sc_mesh(),
        scratch_types=[
            pltpu.VMEM((_SC_SUB, 128), jnp.int32),
            pltpu.VMEM((_SC_HALF, DP), jnp.float32),
            pltpu.SemaphoreType.DMA,
        ],
    )
    def sk(tbl_hbm, dst_hbm, out_hbm, idx_v, buf_v, s1):
        wid = lax.axis_index("s") * NC + lax.axis_index("c")
        base = wid * rows_w

        def body(i, carry):
            off = pl.multiple_of(base + i * _SC_CH, _SC_CH)
            pltpu.sync_copy(
                dst_hbm.at[pl.ds(pl.multiple_of(off // 128, _SC_SUB),
                                 _SC_SUB)], idx_v)
            for half in range(2):
                pltpu.sync_copy(
                    tbl_hbm.at[pl.ds(off + half * _SC_HALF, _SC_HALF)],
                    buf_v)
                cps = []
                for j in range(_SC_SUB // 2):
                    cps.append(pltpu.async_copy(
                        buf_v.at[pl.ds(j * 128, 128)],
                        out_hbm.at[idx_v.at[half * (_SC_SUB // 2) + j]],
                        s1))
                for cp in cps:
                    cp.wait()
            return carry

        lax.fori_loop(0, nch, body, 0)

    return sk(tbl2, dst2)


def _attn_body(sqkv_ref, stc_ref, str_ref, strp_ref, so_ref, *, slab):
    i = pl.program_id(2)
    qr = TILE_CH * BUCKET
    kr = qr + BUCKET
    q0 = pl.multiple_of(i * qr, qr)
    kp = pl.multiple_of(lax.rem(q0 - BUCKET + slab, slab), BUCKET)

    q = sqkv_ref[0, 0, pl.ds(q0, qr), :DH]
    kw = jnp.concatenate([sqkv_ref[0, 0, pl.ds(kp, BUCKET), :DH], q], axis=0)
    vw = jnp.concatenate([sqkv_ref[0, 0, pl.ds(kp, BUCKET), DH:],
                          sqkv_ref[0, 0, pl.ds(q0, qr), DH:]], axis=0)
    tq = stc_ref[0, 0, pl.ds(q0, qr), :]
    tk = jnp.concatenate([strp_ref[0, 0, :, pl.ds(q0, BUCKET)],
                          str_ref[0, 0, :, pl.ds(q0, qr)]], axis=1)

    kn = kw / (jnp.sqrt(jnp.sum(kw * kw, axis=-1, keepdims=True)) + 1e-9)
    dots = lax.dot_general(q.astype(jnp.bfloat16), kn.astype(jnp.bfloat16),
                           (((1,), (1,)), ((), ())),
                           preferred_element_type=jnp.float32)
    dots = dots * (DH ** -0.5)
    ri = lax.broadcasted_iota(jnp.int32, (qr, kr), 0) // BUCKET
    ci = lax.broadcasted_iota(jnp.int32, (qr, kr), 1) // BUCKET
    band = (ci == ri) | (ci == ri + 1)
    dots = jnp.where(tq < tk, -1e9, dots)
    dots = jnp.where(tq == tk, -1e5, dots)
    dots = jnp.where(band, dots, -1e9)
    m = jnp.max(dots, axis=-1, keepdims=True)
    s = jnp.sum(jnp.exp(dots - m), axis=-1, keepdims=True)
    lse = m + jnp.log(s)
    probs = jnp.exp(dots - lse)
    o = jnp.dot(probs.astype(jnp.bfloat16), vw.astype(jnp.bfloat16),
                preferred_element_type=jnp.float32)
    pad = jnp.zeros((qr, DP - DH - 1), jnp.float32)
    so_ref[0, 0, pl.ds(q0, qr), :] = jnp.concatenate([o, lse, pad], axis=1)


def _attn_call(sqkv, st, batch, seq_len):
    slab = N_HASHES * seq_len
    n_tiles = slab // (TILE_CH * BUCKET)
    stc = st.reshape(batch, N_HEADS, slab, 1)
    str_ = st.reshape(batch, N_HEADS, 1, slab)
    strp = jnp.roll(st, BUCKET, axis=-1).reshape(batch, N_HEADS, 1, slab)
    grid = (batch, N_HEADS, n_tiles)
    return pl.pallas_call(
        functools.partial(_attn_body, slab=slab),
        grid=grid,
        in_specs=[
            pl.BlockSpec((1, 1, slab, DP), lambda b, h, i: (b, h, 0, 0)),
            pl.BlockSpec((1, 1, slab, 1), lambda b, h, i: (b, h, 0, 0)),
            pl.BlockSpec((1, 1, 1, slab), lambda b, h, i: (b, h, 0, 0)),
            pl.BlockSpec((1, 1, 1, slab), lambda b, h, i: (b, h, 0, 0)),
        ],
        out_specs=pl.BlockSpec((1, 1, slab, DP), lambda b, h, i: (b, h, 0, 0)),
        out_shape=jax.ShapeDtypeStruct((batch, N_HEADS, slab, DP),
                                       jnp.float32),
    )(sqkv, stc, str_, strp)


def _ffn_body(x_ref, o2_ref, wo_ref, w1_ref, w2_ref,
              c1_ref, c2_ref, g_ref, b_ref, out_ref):
    parts = []
    for hh in range(N_HEADS):
        o0 = o2_ref[0][:, hh * DP:hh * DP + DH]
        o1 = o2_ref[1][:, hh * DP:hh * DP + DH]
        l0 = o2_ref[0][:, hh * DP + DH:hh * DP + DH + 1]
        l1 = o2_ref[1][:, hh * DP + DH:hh * DP + DH + 1]
        m = jnp.maximum(l0, l1)
        e0 = jnp.exp(l0 - m)
        e1 = jnp.exp(l1 - m)
        inv = 1.0 / (e0 + e1)
        parts.append(o0 * (e0 * inv) + o1 * (e1 * inv))
    a = jnp.concatenate(parts, axis=1)
    xp = x_ref[...] + jnp.dot(a.astype(jnp.bfloat16), wo_ref[...],
                              preferred_element_type=jnp.float32)
    mu = jnp.mean(xp, axis=-1, keepdims=True)
    var = jnp.mean((xp - mu) * (xp - mu), axis=-1, keepdims=True)
    h2 = (xp - mu) / jnp.sqrt(var + 1e-5) * g_ref[...] + b_ref[...]
    f = jnp.maximum(
        jnp.dot(h2.astype(jnp.bfloat16), w1_ref[...],
                preferred_element_type=jnp.float32)
        + c1_ref[...], 0.0)
    out_ref[...] = xp + jnp.dot(f.astype(jnp.bfloat16), w2_ref[...],
                                preferred_element_type=jnp.float32) + c2_ref[...]


def _ffn_call(x2, o2, wo, w1, w2, c1, c2, g2, b2):
    bt = x2.shape[0]
    grid = (bt // T_BLK,)
    return pl.pallas_call(
        _ffn_body,
        grid=grid,
        in_specs=[
            pl.BlockSpec((T_BLK, D_MODEL), lambda i: (i, 0)),
            pl.BlockSpec((2, T_BLK, N_HEADS * DP), lambda i: (0, i, 0)),
            pl.BlockSpec((D_MODEL, D_MODEL), lambda i: (0, 0)),
            pl.BlockSpec((D_MODEL, D_FFN), lambda i: (0, 0)),
            pl.BlockSpec((D_FFN, D_MODEL), lambda i: (0, 0)),
            pl.BlockSpec((1, D_FFN), lambda i: (0, 0)),
            pl.BlockSpec((1, D_MODEL), lambda i: (0, 0)),
            pl.BlockSpec((1, D_MODEL), lambda i: (0, 0)),
            pl.BlockSpec((1, D_MODEL), lambda i: (0, 0)),
        ],
        out_specs=pl.BlockSpec((T_BLK, D_MODEL), lambda i: (i, 0)),
        out_shape=jax.ShapeDtypeStruct((bt, D_MODEL), jnp.float32),
    )(x2, o2, wo, w1, w2, c1, c2, g2, b2)


def _lsh_keys(x2, p, rkey, batch, seq_len):
    n_buckets = seq_len // BUCKET
    x = x2.reshape(batch, seq_len, D_MODEL)
    mu = jnp.mean(x, axis=-1, keepdims=True)
    var = jnp.var(x, axis=-1, keepdims=True)
    h = (x - mu) / jnp.sqrt(var + 1e-5) * p['g1'] + p['b1']
    qk = (h @ p['Wqk']).reshape(batch, seq_len, N_HEADS, DH)
    h_out = h
    qk = qk.transpose(0, 2, 1, 3)
    rmat = jax.random.normal(rkey, (DH, N_HASHES, n_buckets // 2),
                             dtype=qk.dtype)
    rot = jnp.einsum('bhtd,dnr->bhtnr', qk, rmat)
    rot = jnp.concatenate([rot, -rot], axis=-1)
    buckets = jnp.argmax(rot, axis=-1)
    buckets = buckets + jnp.arange(N_HASHES) * n_buckets
    buckets = jnp.moveaxis(buckets, 3, 2).reshape(
        batch, N_HEADS, N_HASHES * seq_len)
    ticker = jnp.broadcast_to(
        jnp.arange(N_HASHES * seq_len), buckets.shape)
    return buckets * seq_len + (ticker % seq_len), h_out


def _layer(x2, p, rkey, batch, seq_len):
    bt = batch * seq_len
    slab = N_HASHES * seq_len
    nrows = batch * N_HEADS * slab

    wqk = p['Wqk'].reshape(D_MODEL, N_HEADS, DH)
    wv = p['Wv'].reshape(D_MODEL, N_HEADS, DH)
    wpack = jnp.stack([wqk, wv], axis=2).reshape(
        D_MODEL, N_HEADS * DP).astype(jnp.bfloat16)

    keys_bht, h2 = _lsh_keys(x2, p, rkey, batch, seq_len)
    qkv2 = _qkv_call(h2.reshape(bt, D_MODEL), wpack)
    sticker = jnp.argsort(keys_bht, axis=-1)
    st = (sticker % seq_len).astype(jnp.int32)
    rnd = (sticker // seq_len).astype(jnp.int32)

    b_idx = lax.broadcasted_iota(jnp.int32, (batch, N_HEADS, slab), 0)
    h_idx = lax.broadcasted_iota(jnp.int32, (batch, N_HEADS, slab), 1)
    src = (b_idx * seq_len + st) * N_HEADS + h_idx
    dst = rnd * (bt * N_HEADS) + (b_idx * seq_len + st) * N_HEADS + h_idx

    sqkv_rows = _sc_gather(qkv2.reshape(bt * N_HEADS, DP),
                           src.reshape(nrows // 128, 128))
    sqkv = sqkv_rows.reshape(batch, N_HEADS, slab, DP)

    so = _attn_call(sqkv, st, batch, seq_len)

    o_rows = _sc_scatter(so.reshape(nrows, DP),
                         dst.reshape(nrows // 128, 128))
    o2 = o_rows.reshape(N_HASHES, bt, N_HEADS * DP)

    return _ffn_call(x2, o2,
                     p['Wo'].astype(jnp.bfloat16),
                     p['W1'].astype(jnp.bfloat16),
                     p['W2'].astype(jnp.bfloat16),
                     p['c1'].reshape(1, -1), p['c2'].reshape(1, -1),
                     p['g2'].reshape(1, -1), p['b2'].reshape(1, -1))


def kernel(x, params):
    batch, seq_len, _ = x.shape
    x2 = x.reshape(batch * seq_len, D_MODEL)
    for i, p in enumerate(params):
        rkey = jax.random.fold_in(jax.random.key(42), i)
        x2 = _layer(x2, p, rkey, batch, seq_len)
    return x2.reshape(batch, seq_len, D_MODEL)
